# baseline (speedup 1.0000x reference)
import functools

import jax
import jax.numpy as jnp
from jax import lax
from jax.experimental import pallas as pl
from jax.experimental.pallas import tpu as pltpu
from jax.experimental.pallas import tpu_sc as plsc

B = 16384
F = 26
D = 16
V = 1000000
NUM_CORES = 2
NUM_SUBCORES = 16
NW = NUM_CORES * NUM_SUBCORES
BPW = B // NW

TCOL = 512
TGRID = 1954
TS_ROWS = TGRID * TCOL

CHUNK = 256
NCH = BPW // CHUNK
NSTEPS = F * NCH
NBUF = 4


def _t1_body(in_ref, out_ref):
    y = in_ref[...].T.reshape(TCOL // 8, 8, D)
    for j in range(8):
        out_ref[:, j * D:(j + 1) * D] = y[:, j, :]


_t1 = pl.pallas_call(
    _t1_body,
    grid=(TGRID,),
    in_specs=[pl.BlockSpec((D, TCOL), lambda g: (0, g))],
    out_specs=pl.BlockSpec((TCOL * D // 128, 128), lambda g: (g, 0)),
    out_shape=jax.ShapeDtypeStruct((TS_ROWS * D // 128, 128), jnp.float32),
)


def _t2_body(in_ref, out_ref):
    x = in_ref[...]
    y = jnp.stack([x[:, j * D:(j + 1) * D] for j in range(8)], axis=1)
    out_ref[0] = y.reshape(_T2_BB, D).T


_T2_BB = 512
_T2_RB = _T2_BB * D // 128


_t2 = pl.pallas_call(
    _t2_body,
    grid=(F, B // _T2_BB),
    in_specs=[pl.BlockSpec((_T2_RB, 128),
                           lambda f, g: (f * (B // _T2_BB) + g, 0))],
    out_specs=pl.BlockSpec((1, D, _T2_BB), lambda f, g: (f, 0, g)),
    out_shape=jax.ShapeDtypeStruct((F, D, B), jnp.float32),
)


@functools.partial(
    pl.kernel,
    mesh=plsc.VectorSubcoreMesh(core_axis_name="c", subcore_axis_name="s"),
    out_type=jax.ShapeDtypeStruct((F, B, D), jnp.float32),
    compiler_params=pltpu.CompilerParams(use_tc_tiling_on_sc=False),
    scratch_types=(
        [pltpu.VMEM((F, BPW), jnp.int32)]
        + [pltpu.VMEM((CHUNK, D), jnp.float32) for _ in range(NBUF)]
        + [pltpu.SemaphoreType.DMA for _ in range(2 * NBUF)]
    ),
)
def _gather_kernel(idxT_hbm, ts_hbm, out_hbm, idx_v, *bufs_and_sems):
    bufs = bufs_and_sems[:NBUF]
    sem_g = bufs_and_sems[NBUF:2 * NBUF]
    sem_o = bufs_and_sems[2 * NBUF:]
    wid = lax.axis_index("s") * NUM_CORES + lax.axis_index("c")
    b0 = wid * BPW
    pltpu.sync_copy(idxT_hbm.at[:, pl.ds(b0, BPW)], idx_v)

    def gather(step, slot):
        f, c = divmod(step, NCH)
        return pltpu.async_copy(
            ts_hbm.at[idx_v.at[f, pl.ds(c * CHUNK, CHUNK)]],
            bufs[slot], sem_g[slot])

    def writeback(step, slot):
        f, c = divmod(step, NCH)
        return pltpu.async_copy(
            bufs[slot], out_hbm.at[f, pl.ds(b0 + c * CHUNK, CHUNK)],
            sem_o[slot])

    gathers = [None] * NSTEPS
    outs = [None] * NSTEPS
    for s in range(min(NBUF, NSTEPS)):
        gathers[s] = gather(s, s % NBUF)
    for s in range(NSTEPS):
        gathers[s].wait()
        outs[s] = writeback(s, s % NBUF)
        if s + NBUF < NSTEPS:
            outs[s].wait()
            gathers[s + NBUF] = gather(s + NBUF, s % NBUF)
    for s in range(max(0, NSTEPS - NBUF), NSTEPS):
        outs[s].wait()


def kernel(inputs, table):
    ts = _t1(table.T)
    out2 = _gather_kernel(inputs.T, ts.reshape(TS_ROWS, D))
    out3 = _t2(out2.reshape(F * B * D // 128, 128))
    return out3.transpose(2, 0, 1)


# speedup vs baseline: 2.1968x; 2.1968x over previous
import functools

import jax
import jax.numpy as jnp
from jax import lax
from jax.experimental import pallas as pl
from jax.experimental.pallas import tpu as pltpu
from jax.experimental.pallas import tpu_sc as plsc

B = 16384
F = 26
D = 16
V = 1000000
NUM_CORES = 2
NUM_SUBCORES = 16
NW = NUM_CORES * NUM_SUBCORES
BPW = B // NW

T1BLK = 1024
T1G = -(-V // T1BLK)
TS_UNITS = T1G * T1BLK

NBUF = 4


def _t1_body(in_ref, out_ref):
    x = in_ref[...]
    x3 = x.reshape(D, 8, 128)
    s = jnp.concatenate([x3[:, m, :] for m in range(8)], axis=0)
    out_ref[...] = s.T


_t1 = pl.pallas_call(
    _t1_body,
    grid=(T1G,),
    in_specs=[pl.BlockSpec((D, T1BLK), lambda g: (0, g))],
    out_specs=pl.BlockSpec((128, 128), lambda g: (g, 0)),
    out_shape=jax.ShapeDtypeStruct((T1G * 128, 128), jnp.float32),
)


def _t2_body(in_ref, out_ref):
    w = in_ref[...]
    y = w.T.reshape(8, D, 128)
    for j in range(8):
        out_ref[0, :, j * 128:(j + 1) * 128] = y[j]


_t2 = pl.pallas_call(
    _t2_body,
    grid=(F, B // T1BLK),
    in_specs=[pl.BlockSpec((128, 128),
                           lambda f, c: (f * (B // T1BLK) + c, 0))],
    out_specs=pl.BlockSpec((1, D, T1BLK), lambda f, c: (f, 0, c)),
    out_shape=jax.ShapeDtypeStruct((F, D, B), jnp.float32),
)


def _perm(i):
    return (i & ~1023) | ((i & 127) << 3) | ((i >> 7) & 7)


@functools.partial(
    pl.kernel,
    mesh=plsc.VectorSubcoreMesh(core_axis_name="c", subcore_axis_name="s"),
    out_type=jax.ShapeDtypeStruct((F * B, D), jnp.float32),
    compiler_params=pltpu.CompilerParams(use_tc_tiling_on_sc=False),
    scratch_types=(
        [pltpu.VMEM((F, BPW), jnp.int32),
         pltpu.VMEM((F, BPW), jnp.int32)]
        + [pltpu.VMEM((BPW, D), jnp.float32) for _ in range(NBUF)]
        + [pltpu.SemaphoreType.DMA for _ in range(2 * NBUF)]
    ),
)
def _gather_kernel(gidx_hbm, sidx_hbm, ts_hbm, out_hbm, gidx_v, sidx_v,
                   *bufs_and_sems):
    bufs = bufs_and_sems[:NBUF]
    sem_g = bufs_and_sems[NBUF:2 * NBUF]
    sem_o = bufs_and_sems[2 * NBUF:]
    wid = lax.axis_index("s") * NUM_CORES + lax.axis_index("c")
    b0 = wid * BPW
    pltpu.sync_copy(gidx_hbm.at[:, pl.ds(b0, BPW)], gidx_v)
    pltpu.sync_copy(sidx_hbm.at[:, pl.ds(b0, BPW)], sidx_v)

    def gather(f, slot):
        return pltpu.async_copy(
            ts_hbm.at[gidx_v.at[f]], bufs[slot], sem_g[slot])

    def writeback(f, slot):
        return pltpu.async_copy(
            bufs[slot], out_hbm.at[sidx_v.at[f]], sem_o[slot])

    gathers = [None] * F
    outs = [None] * F
    for f in range(min(NBUF, F)):
        gathers[f] = gather(f, f % NBUF)
    for f in range(F):
        gathers[f].wait()
        outs[f] = writeback(f, f % NBUF)
        if f + NBUF < F:
            outs[f].wait()
            gathers[f + NBUF] = gather(f + NBUF, f % NBUF)
    for f in range(max(0, F - NBUF), F):
        outs[f].wait()


def kernel(inputs, table):
    ts = _t1(table.T)
    gidx = _perm(inputs).T
    barange = _perm(jnp.arange(B, dtype=jnp.int32))
    sidx = barange[None, :] + (jnp.arange(F, dtype=jnp.int32) * B)[:, None]
    out5 = _gather_kernel(gidx, sidx, ts.reshape(TS_UNITS, D))
    out3 = _t2(out5.reshape(F * B * D // 128, 128))
    return out3.transpose(2, 0, 1)


# speedup vs baseline: 8.9521x; 4.0750x over previous
import functools

import jax
import jax.numpy as jnp
from jax import lax
from jax.experimental import pallas as pl
from jax.experimental.pallas import tpu as pltpu
from jax.experimental.pallas import tpu_sc as plsc

B = 16384
F = 26
D = 16
V = 1000000
NUM_CORES = 2
NUM_SUBCORES = 16
NW = NUM_CORES * NUM_SUBCORES
BPW = B // NW

T1BLK = 8192
T1G = -(-V // T1BLK)
TS_UNITS = T1G * T1BLK

NBUF = 4


def _t1_body(in_ref, out_ref):
    x = in_ref[...]
    x3 = x.reshape(D, T1BLK // 128, 128)
    for q in range(T1BLK // 1024):
        s = jnp.concatenate(
            [x3[:, q * 8 + m, :] for m in range(8)], axis=0)
        out_ref[q * 128:(q + 1) * 128, :] = s.T


_t1 = pl.pallas_call(
    _t1_body,
    grid=(T1G,),
    in_specs=[pl.BlockSpec((D, T1BLK), lambda g: (0, g))],
    out_specs=pl.BlockSpec((T1BLK // 8, 128), lambda g: (g, 0)),
    out_shape=jax.ShapeDtypeStruct((T1G * (T1BLK // 8), 128), jnp.float32),
)


_T2BLK = 8192


def _t2_body(in_ref, out_ref):
    x = in_ref[...]
    for q in range(_T2BLK // 1024):
        w = x[q * 128:(q + 1) * 128, :]
        y = w.T.reshape(8, D, 128)
        for j in range(8):
            out_ref[0, :, q * 1024 + j * 128:q * 1024 + (j + 1) * 128] = y[j]


_t2 = pl.pallas_call(
    _t2_body,
    grid=(F, B // _T2BLK),
    in_specs=[pl.BlockSpec((_T2BLK * D // 128, 128),
                           lambda f, c: (f * (B // _T2BLK) + c, 0))],
    out_specs=pl.BlockSpec((1, D, _T2BLK), lambda f, c: (f, 0, c)),
    out_shape=jax.ShapeDtypeStruct((F, D, B), jnp.float32),
)


def _perm(i):
    return (i & ~1023) | ((i & 127) << 3) | ((i >> 7) & 7)


@functools.partial(
    pl.kernel,
    mesh=plsc.VectorSubcoreMesh(core_axis_name="c", subcore_axis_name="s"),
    out_type=jax.ShapeDtypeStruct((F * B, D), jnp.float32),
    compiler_params=pltpu.CompilerParams(use_tc_tiling_on_sc=False),
    scratch_types=(
        [pltpu.VMEM((F, BPW), jnp.int32),
         pltpu.VMEM((F, BPW), jnp.int32)]
        + [pltpu.VMEM((BPW, D), jnp.float32) for _ in range(NBUF)]
        + [pltpu.SemaphoreType.DMA for _ in range(2 * NBUF)]
    ),
)
def _gather_kernel(gidx_hbm, sidx_hbm, ts_hbm, out_hbm, gidx_v, sidx_v,
                   *bufs_and_sems):
    bufs = bufs_and_sems[:NBUF]
    sem_g = bufs_and_sems[NBUF:2 * NBUF]
    sem_o = bufs_and_sems[2 * NBUF:]
    wid = lax.axis_index("s") * NUM_CORES + lax.axis_index("c")
    b0 = wid * BPW
    pltpu.sync_copy(gidx_hbm.at[:, pl.ds(b0, BPW)], gidx_v)
    pltpu.sync_copy(sidx_hbm.at[:, pl.ds(b0, BPW)], sidx_v)

    def gather(f, slot):
        return pltpu.async_copy(
            ts_hbm.at[gidx_v.at[f]], bufs[slot], sem_g[slot])

    def writeback(f, slot):
        return pltpu.async_copy(
            bufs[slot], out_hbm.at[sidx_v.at[f]], sem_o[slot])

    gathers = [None] * F
    outs = [None] * F
    for f in range(min(NBUF, F)):
        gathers[f] = gather(f, f % NBUF)
    for f in range(F):
        gathers[f].wait()
        outs[f] = writeback(f, f % NBUF)
        if f + NBUF < F:
            outs[f].wait()
            gathers[f + NBUF] = gather(f + NBUF, f % NBUF)
    for f in range(max(0, F - NBUF), F):
        outs[f].wait()


def kernel(inputs, table):
    ts = _t1(table.T)
    gidx = _perm(inputs).T
    barange = _perm(jnp.arange(B, dtype=jnp.int32))
    sidx = barange[None, :] + (jnp.arange(F, dtype=jnp.int32) * B)[:, None]
    out5 = _gather_kernel(gidx, sidx, ts.reshape(TS_UNITS, D))
    out3 = _t2(out5.reshape(F * B * D // 128, 128))
    return out3.transpose(2, 0, 1)


# speedup vs baseline: 12.6021x; 1.4077x over previous
import functools

import jax
import jax.numpy as jnp
from jax import lax
from jax.experimental import pallas as pl
from jax.experimental.pallas import tpu as pltpu
from jax.experimental.pallas import tpu_sc as plsc

B = 16384
F = 26
D = 16
V = 1000000
NUM_CORES = 2
NUM_SUBCORES = 16
NW = NUM_CORES * NUM_SUBCORES
BPW = B // NW

T1BLK = 32768
T1G = -(-V // T1BLK)
TS_UNITS = T1G * T1BLK

NBUF = 4


def _t1_body(in_ref, out_ref):
    x = in_ref[...]
    x3 = x.reshape(D, T1BLK // 128, 128)
    for q in range(T1BLK // 1024):
        s = jnp.concatenate(
            [x3[:, q * 8 + m, :] for m in range(8)], axis=0)
        out_ref[q * 128:(q + 1) * 128, :] = s.T


_t1 = pl.pallas_call(
    _t1_body,
    grid=(T1G,),
    in_specs=[pl.BlockSpec((D, T1BLK), lambda g: (0, g))],
    out_specs=pl.BlockSpec((T1BLK // 8, 128), lambda g: (g, 0)),
    out_shape=jax.ShapeDtypeStruct((T1G * (T1BLK // 8), 128), jnp.float32),
)


_T2BLK = 16384


def _t2_body(in_ref, out_ref):
    x = in_ref[...]
    for q in range(_T2BLK // 1024):
        w = x[q * 128:(q + 1) * 128, :]
        y = w.T.reshape(8, D, 128)
        for j in range(8):
            out_ref[0, :, q * 1024 + j * 128:q * 1024 + (j + 1) * 128] = y[j]


_t2 = pl.pallas_call(
    _t2_body,
    grid=(F, B // _T2BLK),
    in_specs=[pl.BlockSpec((_T2BLK * D // 128, 128),
                           lambda f, c: (f * (B // _T2BLK) + c, 0))],
    out_specs=pl.BlockSpec((1, D, _T2BLK), lambda f, c: (f, 0, c)),
    out_shape=jax.ShapeDtypeStruct((F, D, B), jnp.float32),
)


def _perm(i):
    return (i & ~1023) | ((i & 127) << 3) | ((i >> 7) & 7)


@functools.partial(
    pl.kernel,
    mesh=plsc.VectorSubcoreMesh(core_axis_name="c", subcore_axis_name="s"),
    out_type=jax.ShapeDtypeStruct((F * B, D), jnp.float32),
    compiler_params=pltpu.CompilerParams(use_tc_tiling_on_sc=False),
    scratch_types=(
        [pltpu.VMEM((F, BPW), jnp.int32),
         pltpu.VMEM((F, BPW), jnp.int32)]
        + [pltpu.VMEM((BPW, D), jnp.float32) for _ in range(NBUF)]
        + [pltpu.SemaphoreType.DMA for _ in range(2 * NBUF)]
    ),
)
def _gather_kernel(gidx_hbm, sidx_hbm, ts_hbm, out_hbm, gidx_v, sidx_v,
                   *bufs_and_sems):
    bufs = bufs_and_sems[:NBUF]
    sem_g = bufs_and_sems[NBUF:2 * NBUF]
    sem_o = bufs_and_sems[2 * NBUF:]
    wid = lax.axis_index("s") * NUM_CORES + lax.axis_index("c")
    b0 = wid * BPW
    pltpu.sync_copy(gidx_hbm.at[:, pl.ds(b0, BPW)], gidx_v)
    pltpu.sync_copy(sidx_hbm.at[:, pl.ds(b0, BPW)], sidx_v)

    def gather(f, slot):
        return pltpu.async_copy(
            ts_hbm.at[gidx_v.at[f]], bufs[slot], sem_g[slot])

    def writeback(f, slot):
        return pltpu.async_copy(
            bufs[slot], out_hbm.at[sidx_v.at[f]], sem_o[slot])

    gathers = [None] * F
    outs = [None] * F
    for f in range(min(NBUF, F)):
        gathers[f] = gather(f, f % NBUF)
    for f in range(F):
        gathers[f].wait()
        outs[f] = writeback(f, f % NBUF)
        if f + NBUF < F:
            outs[f].wait()
            gathers[f + NBUF] = gather(f + NBUF, f % NBUF)
    for f in range(max(0, F - NBUF), F):
        outs[f].wait()


def kernel(inputs, table):
    ts = _t1(table.T)
    gidx = _perm(inputs).T
    barange = _perm(jnp.arange(B, dtype=jnp.int32))
    sidx = barange[None, :] + (jnp.arange(F, dtype=jnp.int32) * B)[:, None]
    out5 = _gather_kernel(gidx, sidx, ts.reshape(TS_UNITS, D))
    out3 = _t2(out5.reshape(F * B * D // 128, 128))
    return out3.transpose(2, 0, 1)


# speedup vs baseline: 13.1782x; 1.0457x over previous
import functools

import jax
import jax.numpy as jnp
from jax import lax
from jax.experimental import pallas as pl
from jax.experimental.pallas import tpu as pltpu
from jax.experimental.pallas import tpu_sc as plsc

B = 16384
F = 26
D = 16
V = 1000000
NUM_CORES = 2
NUM_SUBCORES = 16
NW = NUM_CORES * NUM_SUBCORES
BPW = B // NW

T1BLK = 65536
T1G = -(-V // T1BLK)
TS_UNITS = T1G * T1BLK

NBUF = 4


def _t1_body(in_ref, out_ref):
    x = in_ref[...]
    x3 = x.reshape(D, T1BLK // 128, 128)
    for q in range(T1BLK // 1024):
        s = jnp.concatenate(
            [x3[:, q * 8 + m, :] for m in range(8)], axis=0)
        out_ref[q * 128:(q + 1) * 128, :] = s.T


_t1 = pl.pallas_call(
    _t1_body,
    grid=(T1G,),
    in_specs=[pl.BlockSpec((D, T1BLK), lambda g: (0, g))],
    out_specs=pl.BlockSpec((T1BLK // 8, 128), lambda g: (g, 0)),
    out_shape=jax.ShapeDtypeStruct((T1G * (T1BLK // 8), 128), jnp.float32),
)


_T2BLK = 16384


def _t2_body(in_ref, out_ref):
    x = in_ref[...]
    for q in range(_T2BLK // 1024):
        w = x[q * 128:(q + 1) * 128, :]
        y = w.T.reshape(8, D, 128)
        for j in range(8):
            out_ref[0, :, q * 1024 + j * 128:q * 1024 + (j + 1) * 128] = y[j]


_t2 = pl.pallas_call(
    _t2_body,
    grid=(F, B // _T2BLK),
    in_specs=[pl.BlockSpec((_T2BLK * D // 128, 128),
                           lambda f, c: (f * (B // _T2BLK) + c, 0))],
    out_specs=pl.BlockSpec((1, D, _T2BLK), lambda f, c: (f, 0, c)),
    out_shape=jax.ShapeDtypeStruct((F, D, B), jnp.float32),
)


def _perm(i):
    return (i & ~1023) | ((i & 127) << 3) | ((i >> 7) & 7)


@functools.partial(
    pl.kernel,
    mesh=plsc.VectorSubcoreMesh(core_axis_name="c", subcore_axis_name="s"),
    out_type=jax.ShapeDtypeStruct((F * B, D), jnp.float32),
    compiler_params=pltpu.CompilerParams(use_tc_tiling_on_sc=False),
    scratch_types=(
        [pltpu.VMEM((F, BPW), jnp.int32),
         pltpu.VMEM((F, BPW), jnp.int32)]
        + [pltpu.VMEM((BPW, D), jnp.float32) for _ in range(NBUF)]
        + [pltpu.SemaphoreType.DMA for _ in range(2 * NBUF)]
    ),
)
def _gather_kernel(gidx_hbm, sidx_hbm, ts_hbm, out_hbm, gidx_v, sidx_v,
                   *bufs_and_sems):
    bufs = bufs_and_sems[:NBUF]
    sem_g = bufs_and_sems[NBUF:2 * NBUF]
    sem_o = bufs_and_sems[2 * NBUF:]
    wid = lax.axis_index("s") * NUM_CORES + lax.axis_index("c")
    b0 = wid * BPW
    pltpu.sync_copy(gidx_hbm.at[:, pl.ds(b0, BPW)], gidx_v)
    pltpu.sync_copy(sidx_hbm.at[:, pl.ds(b0, BPW)], sidx_v)

    def gather(f, slot):
        return pltpu.async_copy(
            ts_hbm.at[gidx_v.at[f]], bufs[slot], sem_g[slot])

    def writeback(f, slot):
        return pltpu.async_copy(
            bufs[slot], out_hbm.at[sidx_v.at[f]], sem_o[slot])

    gathers = [None] * F
    outs = [None] * F
    for f in range(min(NBUF, F)):
        gathers[f] = gather(f, f % NBUF)
    for f in range(F):
        gathers[f].wait()
        outs[f] = writeback(f, f % NBUF)
        if f + NBUF < F:
            outs[f].wait()
            gathers[f + NBUF] = gather(f + NBUF, f % NBUF)
    for f in range(max(0, F - NBUF), F):
        outs[f].wait()


def kernel(inputs, table):
    ts = _t1(table.T)
    gidx = _perm(inputs).T
    barange = _perm(jnp.arange(B, dtype=jnp.int32))
    sidx = barange[None, :] + (jnp.arange(F, dtype=jnp.int32) * B)[:, None]
    out5 = _gather_kernel(gidx, sidx, ts.reshape(TS_UNITS, D))
    out3 = _t2(out5.reshape(F * B * D // 128, 128))
    return out3.transpose(2, 0, 1)


# speedup vs baseline: 13.4827x; 1.0231x over previous
import functools

import jax
import jax.numpy as jnp
from jax import lax
from jax.experimental import pallas as pl
from jax.experimental.pallas import tpu as pltpu
from jax.experimental.pallas import tpu_sc as plsc

B = 16384
F = 26
D = 16
V = 1000000
NUM_CORES = 2
NUM_SUBCORES = 16
NW = NUM_CORES * NUM_SUBCORES
BPW = B // NW

T1BLK = 131072
T1G = -(-V // T1BLK)
TS_UNITS = T1G * T1BLK

NBUF = 4


def _t1_body(in_ref, out_ref):
    x = in_ref[...]
    x3 = x.reshape(D, T1BLK // 128, 128)
    for q in range(T1BLK // 1024):
        s = jnp.concatenate(
            [x3[:, q * 8 + m, :] for m in range(8)], axis=0)
        out_ref[q * 128:(q + 1) * 128, :] = s.T


_t1 = pl.pallas_call(
    _t1_body,
    grid=(T1G,),
    in_specs=[pl.BlockSpec((D, T1BLK), lambda g: (0, g))],
    out_specs=pl.BlockSpec((T1BLK // 8, 128), lambda g: (g, 0)),
    out_shape=jax.ShapeDtypeStruct((T1G * (T1BLK // 8), 128), jnp.float32),
)


_T2BLK = 16384


def _t2_body(in_ref, out_ref):
    x = in_ref[...]
    for q in range(_T2BLK // 1024):
        w = x[q * 128:(q + 1) * 128, :]
        y = w.T.reshape(8, D, 128)
        for j in range(8):
            out_ref[0, :, q * 1024 + j * 128:q * 1024 + (j + 1) * 128] = y[j]


_t2 = pl.pallas_call(
    _t2_body,
    grid=(F, B // _T2BLK),
    in_specs=[pl.BlockSpec((_T2BLK * D // 128, 128),
                           lambda f, c: (f * (B // _T2BLK) + c, 0))],
    out_specs=pl.BlockSpec((1, D, _T2BLK), lambda f, c: (f, 0, c)),
    out_shape=jax.ShapeDtypeStruct((F, D, B), jnp.float32),
)


def _perm(i):
    return (i & ~1023) | ((i & 127) << 3) | ((i >> 7) & 7)


@functools.partial(
    pl.kernel,
    mesh=plsc.VectorSubcoreMesh(core_axis_name="c", subcore_axis_name="s"),
    out_type=jax.ShapeDtypeStruct((F * B, D), jnp.float32),
    compiler_params=pltpu.CompilerParams(use_tc_tiling_on_sc=False),
    scratch_types=(
        [pltpu.VMEM((F, BPW), jnp.int32),
         pltpu.VMEM((F, BPW), jnp.int32)]
        + [pltpu.VMEM((BPW, D), jnp.float32) for _ in range(NBUF)]
        + [pltpu.SemaphoreType.DMA for _ in range(2 * NBUF)]
    ),
)
def _gather_kernel(gidx_hbm, sidx_hbm, ts_hbm, out_hbm, gidx_v, sidx_v,
                   *bufs_and_sems):
    bufs = bufs_and_sems[:NBUF]
    sem_g = bufs_and_sems[NBUF:2 * NBUF]
    sem_o = bufs_and_sems[2 * NBUF:]
    wid = lax.axis_index("s") * NUM_CORES + lax.axis_index("c")
    b0 = wid * BPW
    pltpu.sync_copy(gidx_hbm.at[:, pl.ds(b0, BPW)], gidx_v)
    pltpu.sync_copy(sidx_hbm.at[:, pl.ds(b0, BPW)], sidx_v)

    def gather(f, slot):
        return pltpu.async_copy(
            ts_hbm.at[gidx_v.at[f]], bufs[slot], sem_g[slot])

    def writeback(f, slot):
        return pltpu.async_copy(
            bufs[slot], out_hbm.at[sidx_v.at[f]], sem_o[slot])

    gathers = [None] * F
    outs = [None] * F
    for f in range(min(NBUF, F)):
        gathers[f] = gather(f, f % NBUF)
    for f in range(F):
        gathers[f].wait()
        outs[f] = writeback(f, f % NBUF)
        if f + NBUF < F:
            outs[f].wait()
            gathers[f + NBUF] = gather(f + NBUF, f % NBUF)
    for f in range(max(0, F - NBUF), F):
        outs[f].wait()


def kernel(inputs, table):
    ts = _t1(table.T)
    gidx = _perm(inputs).T
    barange = _perm(jnp.arange(B, dtype=jnp.int32))
    sidx = barange[None, :] + (jnp.arange(F, dtype=jnp.int32) * B)[:, None]
    out5 = _gather_kernel(gidx, sidx, ts.reshape(TS_UNITS, D))
    out3 = _t2(out5.reshape(F * B * D // 128, 128))
    return out3.transpose(2, 0, 1)
